# Initial kernel scaffold; baseline (speedup 1.0000x reference)
import jax
import jax.numpy as jnp
from jax.experimental import pallas as pl


def kernel(node_feats, edge_index, edge_feats, Wq, bq, Wk, bk, Wv, bv, We, be, Wm, bm, W1, b1, g1, bln1, W2, b2, g2, bln2):
    raise NotImplementedError("write your pallas kernel here")


# speedup vs baseline: 2.0687x; 2.0687x over previous
import functools

import jax
import jax.numpy as jnp
from jax import lax
from jax.experimental import pallas as pl
from jax.experimental.pallas import tpu as pltpu
from jax.experimental.pallas import tpu_sc as plsc

_N = 10000
_E = 320000
_D = 128
_H = 64

_NC = 2
_NS = 16
_C = 80
_EPT = _E // _NS
_NCH = _EPT // _C
_ZCH = _N // _C

_BE = 1280
_BN = 1000


def _edge_proj_body(ef_ref, We_ref, be_ref, Wm_ref, bm_ref,
                    em_ref, maxe_ref, maxm_ref, acc_e, acc_m):
    i = pl.program_id(0)
    ef = ef_ref[...]
    e = jnp.dot(ef, We_ref[...], preferred_element_type=jnp.float32) + be_ref[...]
    m = jnp.dot(ef, Wm_ref[...], preferred_element_type=jnp.float32) + bm_ref[...]
    em_ref[0] = jnp.concatenate([e[:, :_H], m[:, :_H]], axis=1)
    em_ref[1] = jnp.concatenate([e[:, _H:], m[:, _H:]], axis=1)
    ae = jnp.max(jnp.abs(e), axis=0, keepdims=True)
    am = jnp.max(jnp.abs(m), axis=0, keepdims=True)

    @pl.when(i == 0)
    def _():
        acc_e[...] = ae
        acc_m[...] = am

    @pl.when(i > 0)
    def _():
        acc_e[...] = jnp.maximum(acc_e[...], ae)
        acc_m[...] = jnp.maximum(acc_m[...], am)

    @pl.when(i == pl.num_programs(0) - 1)
    def _():
        maxe_ref[...] = acc_e[...]
        maxm_ref[...] = acc_m[...]


def _edge_proj(edge_feats, We, be, Wm, bm):
    return pl.pallas_call(
        _edge_proj_body,
        grid=(_E // _BE,),
        in_specs=[
            pl.BlockSpec((_BE, _D), lambda i: (i, 0)),
            pl.BlockSpec((_D, _D), lambda i: (0, 0)),
            pl.BlockSpec((1, _D), lambda i: (0, 0)),
            pl.BlockSpec((_D, _D), lambda i: (0, 0)),
            pl.BlockSpec((1, _D), lambda i: (0, 0)),
        ],
        out_specs=[
            pl.BlockSpec((2, _BE, _D), lambda i: (0, i, 0)),
            pl.BlockSpec((1, _D), lambda i: (0, 0)),
            pl.BlockSpec((1, _D), lambda i: (0, 0)),
        ],
        out_shape=[
            jax.ShapeDtypeStruct((2, _E, _D), jnp.float32),
            jax.ShapeDtypeStruct((1, _D), jnp.float32),
            jax.ShapeDtypeStruct((1, _D), jnp.float32),
        ],
        scratch_shapes=[
            pltpu.VMEM((1, _D), jnp.float32),
            pltpu.VMEM((1, _D), jnp.float32),
        ],
    )(edge_feats, We, be.reshape(1, _D), Wm, bm.reshape(1, _D))


def _node_proj_body(nf_ref, Wq_ref, bq_ref, Wk_ref, bk_ref, Wv_ref, bv_ref,
                    qv_ref, k2_ref, maxq_ref, maxk_ref, acc_q, acc_k):
    i = pl.program_id(0)
    nf = nf_ref[...]
    q = jnp.dot(nf, Wq_ref[...], preferred_element_type=jnp.float32) + bq_ref[...]
    k = jnp.dot(nf, Wk_ref[...], preferred_element_type=jnp.float32) + bk_ref[...]
    v = jnp.dot(nf, Wv_ref[...], preferred_element_type=jnp.float32) + bv_ref[...]
    qv_ref[0] = jnp.concatenate([q[:, :_H], v[:, :_H]], axis=1)
    qv_ref[1] = jnp.concatenate([q[:, _H:], v[:, _H:]], axis=1)
    k2_ref[...] = k
    aq = jnp.max(jnp.abs(q), axis=0, keepdims=True)
    ak = jnp.max(jnp.abs(k), axis=0, keepdims=True)

    @pl.when(i == 0)
    def _():
        acc_q[...] = aq
        acc_k[...] = ak

    @pl.when(i > 0)
    def _():
        acc_q[...] = jnp.maximum(acc_q[...], aq)
        acc_k[...] = jnp.maximum(acc_k[...], ak)

    @pl.when(i == pl.num_programs(0) - 1)
    def _():
        maxq_ref[...] = acc_q[...]
        maxk_ref[...] = acc_k[...]


def _node_proj(node_feats, Wq, bq, Wk, bk, Wv, bv):
    return pl.pallas_call(
        _node_proj_body,
        grid=(_N // _BN,),
        in_specs=[
            pl.BlockSpec((_BN, _D), lambda i: (i, 0)),
            pl.BlockSpec((_D, _D), lambda i: (0, 0)),
            pl.BlockSpec((1, _D), lambda i: (0, 0)),
            pl.BlockSpec((_D, _D), lambda i: (0, 0)),
            pl.BlockSpec((1, _D), lambda i: (0, 0)),
            pl.BlockSpec((_D, _D), lambda i: (0, 0)),
            pl.BlockSpec((1, _D), lambda i: (0, 0)),
        ],
        out_specs=[
            pl.BlockSpec((2, _BN, _D), lambda i: (0, i, 0)),
            pl.BlockSpec((_BN, _D), lambda i: (i, 0)),
            pl.BlockSpec((1, _D), lambda i: (0, 0)),
            pl.BlockSpec((1, _D), lambda i: (0, 0)),
        ],
        out_shape=[
            jax.ShapeDtypeStruct((2, _N, _D), jnp.float32),
            jax.ShapeDtypeStruct((_N, _D), jnp.float32),
            jax.ShapeDtypeStruct((1, _D), jnp.float32),
            jax.ShapeDtypeStruct((1, _D), jnp.float32),
        ],
        scratch_shapes=[
            pltpu.VMEM((1, _D), jnp.float32),
            pltpu.VMEM((1, _D), jnp.float32),
        ],
    )(node_feats, Wq, bq.reshape(1, _D), Wk, bk.reshape(1, _D), Wv,
      bv.reshape(1, _D))


def _sc_edge_body(qv_hbm, k_hbm, em_hbm, src_hbm, dst_hbm, bnd_hbm,
                  acc_hbm,
                  acc_sh, idx_src, idx_dst, idx_g, qv_buf, k_buf, em_buf,
                  contrib, bnd_buf, sem):
    c = lax.axis_index("c")
    s = lax.axis_index("s")
    zeros16 = jnp.zeros((16,), jnp.float32)

    def _zrow(r, carry):
        for j in range(_D // 16):
            contrib[r, pl.ds(j * 16, 16)] = zeros16
        return carry
    lax.fori_loop(0, _C, _zrow, 0)

    def _zchunk(t, carry):
        jj = t * _NS + s

        @pl.when(jj < _ZCH)
        def _():
            pltpu.sync_copy(contrib, acc_sh.at[pl.ds(jj * _C, _C)])
        return carry
    lax.fori_loop(0, (_ZCH + _NS - 1) // _NS, _zchunk, 0)
    plsc.subcore_barrier()

    pltpu.sync_copy(bnd_hbm.at[pl.ds(c * _H, _H)], bnd_buf)
    bnd = [bnd_buf[pl.ds(j * 16, 16)] for j in range(_H // 16)]
    coff = c * _N
    koff = c * _H

    def _chunk(i, carry):
        base = s * _EPT + i * _C
        pltpu.sync_copy(src_hbm.at[pl.ds(base, _C)], idx_src)
        pltpu.sync_copy(dst_hbm.at[pl.ds(base, _C)], idx_dst)

        def _adj(t, carry2):
            idx_g[pl.ds(t * 16, 16)] = idx_src[pl.ds(t * 16, 16)] + coff
            return carry2
        lax.fori_loop(0, _C // 16, _adj, 0)
        pltpu.async_copy(qv_hbm.at[idx_g], qv_buf, sem).wait()

        pltpu.async_copy(k_hbm.at[idx_dst], k_buf, sem).wait()

        pltpu.sync_copy(em_hbm.at[pl.ds(c * _E + base, _C)], em_buf)

        def _row(r, carry2):
            for j in range(_H // 16):
                qs = qv_buf[r, pl.ds(j * 16, 16)]
                vs = qv_buf[r, pl.ds(_H + j * 16, 16)]
                kd = k_buf[r, pl.ds(koff + j * 16, 16)]
                ee = em_buf[r, pl.ds(j * 16, 16)]
                mm = em_buf[r, pl.ds(_H + j * 16, 16)]
                z = jnp.exp(mm * (qs - kd) + ee - bnd[j])
                contrib[r, pl.ds(j * 16, 16)] = vs * z
                contrib[r, pl.ds(_H + j * 16, 16)] = z
            return carry2
        lax.fori_loop(0, _C, _row, 0)

        pltpu.sync_copy(contrib, acc_sh.at[idx_dst], add=True)
        return carry
    lax.fori_loop(0, _NCH, _chunk, 0)
    plsc.subcore_barrier()

    def _dchunk(t, carry):
        jj = t * _NS + s

        @pl.when(jj < _ZCH)
        def _():
            pltpu.sync_copy(acc_sh.at[pl.ds(jj * _C, _C)], contrib)
            pltpu.sync_copy(contrib, acc_hbm.at[pl.ds(coff + jj * _C, _C)])
        return carry
    lax.fori_loop(0, (_ZCH + _NS - 1) // _NS, _dchunk, 0)


def _sc_edge(qv_flat, k_flat, em_flat, src, dst, bound):
    mesh = plsc.VectorSubcoreMesh(core_axis_name="c", subcore_axis_name="s",
                                  num_cores=_NC, num_subcores=_NS)
    return pl.kernel(
        _sc_edge_body,
        out_type=jax.ShapeDtypeStruct((_NC * _N, _D), jnp.float32),
        mesh=mesh,
        scratch_types=[
            pltpu.VMEM_SHARED((_N, _D), jnp.float32),
            pltpu.VMEM((_C,), jnp.int32),
            pltpu.VMEM((_C,), jnp.int32),
            pltpu.VMEM((_C,), jnp.int32),
            pltpu.VMEM((_C, _D), jnp.float32),
            pltpu.VMEM((_C, _D), jnp.float32),
            pltpu.VMEM((_C, _D), jnp.float32),
            pltpu.VMEM((_C, _D), jnp.float32),
            pltpu.VMEM((_H,), jnp.float32),
            pltpu.SemaphoreType.DMA,
        ],
    )(qv_flat, k_flat, em_flat, src, dst, bound)


def _final_body(a0_ref, a1_ref, nf_ref, W1_ref, b1_ref, g1_ref, bl1_ref,
                W2_ref, b2_ref, g2_ref, bl2_ref, out_ref):
    a0 = a0_ref[0]
    a1 = a1_ref[0]
    num = jnp.concatenate([a0[:, :_H], a1[:, :_H]], axis=1)
    den = jnp.concatenate([a0[:, _H:], a1[:, _H:]], axis=1)
    safe = jnp.where(den > 0.0, den, 1.0)
    h = jnp.where(den > 0.0, num / safe, 0.0)
    x = h + nf_ref[...]
    t = jnp.dot(x, W1_ref[...], preferred_element_type=jnp.float32) + b1_ref[...]
    t = t * jnp.tanh(jax.nn.softplus(t))
    mu = jnp.mean(t, axis=-1, keepdims=True)
    var = jnp.var(t, axis=-1, keepdims=True)
    t = (t - mu) / jnp.sqrt(var + 1e-5) * g1_ref[...] + bl1_ref[...]
    y = jnp.dot(t, W2_ref[...], preferred_element_type=jnp.float32) + b2_ref[...]
    mu2 = jnp.mean(y, axis=-1, keepdims=True)
    var2 = jnp.var(y, axis=-1, keepdims=True)
    out_ref[...] = (y - mu2) / jnp.sqrt(var2 + 1e-5) * g2_ref[...] + bl2_ref[...]


def _final(acc3, node_feats, W1, b1, g1, bl1, W2, b2, g2, bl2):
    full = lambda i: (0, 0)
    return pl.pallas_call(
        _final_body,
        grid=(_N // _BN,),
        in_specs=[
            pl.BlockSpec((1, _BN, _D), lambda i: (0, i, 0)),
            pl.BlockSpec((1, _BN, _D), lambda i: (1, i, 0)),
            pl.BlockSpec((_BN, _D), lambda i: (i, 0)),
            pl.BlockSpec((_D, _D), full),
            pl.BlockSpec((1, _D), full),
            pl.BlockSpec((1, _D), full),
            pl.BlockSpec((1, _D), full),
            pl.BlockSpec((_D, _D), full),
            pl.BlockSpec((1, _D), full),
            pl.BlockSpec((1, _D), full),
            pl.BlockSpec((1, _D), full),
        ],
        out_specs=pl.BlockSpec((_BN, _D), lambda i: (i, 0)),
        out_shape=jax.ShapeDtypeStruct((_N, _D), jnp.float32),
    )(acc3, acc3, node_feats, W1, b1.reshape(1, _D), g1.reshape(1, _D),
      bl1.reshape(1, _D), W2, b2.reshape(1, _D), g2.reshape(1, _D),
      bl2.reshape(1, _D))


def kernel(node_feats, edge_index, edge_feats, Wq, bq, Wk, bk, Wv, bv, We, be,
           Wm, bm, W1, b1, g1, bln1, W2, b2, g2, bln2):
    em, maxe, maxm = _edge_proj(edge_feats, We, be, Wm, bm)
    qv, k2, maxq, maxk = _node_proj(node_feats, Wq, bq, Wk, bk, Wv, bv)

    bound = (maxm * (maxq + maxk) + maxe).reshape(_D)

    src = edge_index[0]
    dst = edge_index[1]
    acc = _sc_edge(qv.reshape(_NC * _N, _D), k2,
                   em.reshape(_NC * _E, _D), src, dst, bound)

    h_out = _final(acc.reshape(_NC, _N, _D), node_feats,
                   W1, b1, g1, bln1, W2, b2, g2, bln2)
    return (h_out, edge_feats)
